# baseline (speedup 1.0000x reference)
import functools

import jax
import jax.numpy as jnp
from jax import lax
from jax.experimental import pallas as pl
from jax.experimental.pallas import tpu as pltpu
from jax.experimental.pallas import tpu_sc as plsc

N_NODES = 10000
N_INC = 320000
D_FEAT = 128
N_ACC = 10112
ROWS_PER_TILE = N_ACC // 16
NW = 32
CHUNK = 128
NC_CORE = (107, 50)
MAXCH = max(NC_CORE)
TOTCH = 16 * (NC_CORE[0] + NC_CORE[1])
HIST_PER_W = 10240
GROUPS = [[1, 2, 3, 4, 5, 6, 7, 8],
          [1001, 1002, 1003, 1004, 1005, 1006, 1007, 1008],
          [2001, 2002, 2003, 2004, 2005, 2006, 2007, 2008],
          [3001, 3002, 3003, 3004, 3005, 3006, 3007, 3008],
          [4001, 4002, 4003, 4004, 4005, 4006, 4007, 4008],
          [5001, 5002, 5003, 5004, 5005, 5006, 5007, 5008],
          [6001, 6002, 6003, 6004, 6005, 6006, 6007, 6008],
          [7001, 7002, 7003, 7004, 7005, 7006, 7007, 7008]]
TARGETS = [g - 1 for grp in GROUPS for g in grp]

_MESH = plsc.VectorSubcoreMesh(core_axis_name="c", subcore_axis_name="s")


@functools.partial(
    pl.kernel,
    mesh=_MESH,
    out_type=jax.ShapeDtypeStruct((2, 16, 2, N_ACC), jnp.float32),
    scratch_types=[
        pltpu.VMEM((HIST_PER_W,), jnp.int32),
        pltpu.VMEM((HIST_PER_W,), jnp.int32),
        pltpu.VMEM((N_ACC,), jnp.float32),
        pltpu.VMEM((N_ACC,), jnp.float32),
    ],
    compiler_params=pltpu.CompilerParams(needs_layout_passes=False),
)
def _hist_pass(nidx, hidx, out, nidx_v, hidx_v, hn, he):
    c = lax.axis_index("c")
    s = lax.axis_index("s")
    w = s * 2 + c
    pltpu.sync_copy(nidx.at[w], nidx_v)
    pltpu.sync_copy(hidx.at[w], hidx_v)

    def zbody(i, carry):
        z = jnp.zeros((16,), jnp.float32)
        hn[pl.ds(16 * i, 16)] = z
        he[pl.ds(16 * i, 16)] = z
        return carry

    lax.fori_loop(0, N_ACC // 16, zbody, 0)

    def hbody(i, carry):
        v = nidx_v[pl.ds(16 * i, 16)]
        cnt, last = plsc.scan_count(v)
        plsc.addupdate_scatter(hn, [v], cnt.astype(jnp.float32), mask=last)
        u = hidx_v[pl.ds(16 * i, 16)]
        cnt2, last2 = plsc.scan_count(u)
        plsc.addupdate_scatter(he, [u], cnt2.astype(jnp.float32), mask=last2)
        return carry

    lax.fori_loop(0, HIST_PER_W // 16, hbody, 0)
    pltpu.sync_copy(hn, out.at[c, s, 0])
    pltpu.sync_copy(he, out.at[c, s, 1])


def _inv_body(h_ref, o_ref):
    t = jnp.sum(h_ref[...], axis=(0, 1))
    o_ref[...] = jnp.where(t > 0, 1.0 / jnp.where(t > 0, t, 1.0), 0.0)


def _inverse_degrees(hist):
    return pl.pallas_call(
        _inv_body,
        out_shape=jax.ShapeDtypeStruct((2, N_ACC), jnp.float32),
    )(hist)


def _mm_body(x_ref, w_ref, o_ref):
    o_ref[...] = jnp.dot(x_ref[...], w_ref[...],
                         preferred_element_type=jnp.float32)


def _matmul(x, W):
    blk = 1000
    return pl.pallas_call(
        _mm_body,
        grid=(N_NODES // blk,),
        in_specs=[pl.BlockSpec((blk, D_FEAT), lambda i: (i, 0)),
                  pl.BlockSpec((D_FEAT, D_FEAT), lambda i: (0, 0))],
        out_specs=pl.BlockSpec((blk, D_FEAT), lambda i: (i, 0)),
        out_shape=jax.ShapeDtypeStruct((N_NODES, D_FEAT), jnp.float32),
    )(x, W)


@functools.partial(
    pl.kernel,
    mesh=_MESH,
    out_type=jax.ShapeDtypeStruct((2, N_ACC, D_FEAT), jnp.float32),
    scratch_types=[
        pltpu.VMEM((MAXCH, CHUNK), jnp.int32),
        pltpu.VMEM((MAXCH, CHUNK), jnp.int32),
        pltpu.VMEM((CHUNK, D_FEAT), jnp.float32),
        pltpu.VMEM_SHARED((N_ACC, D_FEAT), jnp.float32),
        pltpu.SemaphoreType.DMA,
    ],
)
def _segsum_pass(src, idxg, idxs, zeros, out, idxg_v, idxs_v, rows, acc,
                 gsem):
    c = lax.axis_index("c")
    s = lax.axis_index("s")
    w = s * 2 + c
    pltpu.sync_copy(zeros.at[pl.ds(s * ROWS_PER_TILE, ROWS_PER_TILE)],
                    acc.at[pl.ds(s * ROWS_PER_TILE, ROWS_PER_TILE)])
    pltpu.sync_copy(idxg.at[w], idxg_v)
    pltpu.sync_copy(idxs.at[w], idxs_v)
    plsc.subcore_barrier()

    def body(j, carry):
        pltpu.async_copy(src.at[idxg_v.at[j]], rows, gsem).wait()
        pltpu.sync_copy(rows, acc.at[idxs_v.at[j]], add=True)
        return carry

    nmine = jnp.where(c == 0, NC_CORE[0], NC_CORE[1])
    lax.fori_loop(0, nmine, body, 0)
    plsc.subcore_barrier()
    pltpu.sync_copy(acc.at[pl.ds(s * ROWS_PER_TILE, ROWS_PER_TILE)],
                    out.at[c, pl.ds(s * ROWS_PER_TILE, ROWS_PER_TILE)])


TCAPCH = 79


@functools.partial(
    pl.kernel,
    mesh=_MESH,
    out_type=jax.ShapeDtypeStruct((2, 128, D_FEAT), jnp.float32),
    scratch_types=[
        pltpu.VMEM((TCAPCH, CHUNK), jnp.int32),
        pltpu.VMEM((TCAPCH, CHUNK), jnp.int32),
        pltpu.VMEM((NW,), jnp.int32),
        pltpu.VMEM((CHUNK, D_FEAT), jnp.float32),
        pltpu.VMEM_SHARED((128, D_FEAT), jnp.float32),
        pltpu.SemaphoreType.DMA,
    ],
    compiler_params=pltpu.CompilerParams(needs_layout_passes=False),
)
def _segsum_tgt_pass(src, idxg, idxs, trips, zeros, out, idxg_v, idxs_v,
                     tw_v, rows, acc, gsem):
    c = lax.axis_index("c")
    s = lax.axis_index("s")
    w = s * 2 + c
    pltpu.sync_copy(zeros.at[pl.ds(s * 8, 8)], acc.at[pl.ds(s * 8, 8)])
    pltpu.sync_copy(idxg.at[w], idxg_v)
    pltpu.sync_copy(idxs.at[w], idxs_v)
    pltpu.sync_copy(trips, tw_v)
    plsc.subcore_barrier()

    whi = w // 16
    lane = w - whi * 16
    tv = tw_v[pl.ds(whi * 16, 16)]
    nmine = jnp.sum(jnp.where(jnp.arange(16, dtype=jnp.int32) == lane, tv, 0))

    def body(j, carry):
        pltpu.async_copy(src.at[idxg_v.at[j]], rows, gsem).wait()
        pltpu.sync_copy(rows, acc.at[idxs_v.at[j]], add=True)
        return carry

    lax.fori_loop(0, nmine, body, 0)
    plsc.subcore_barrier()
    pltpu.sync_copy(acc.at[pl.ds(s * 8, 8)], out.at[c, pl.ds(s * 8, 8)])


def _fin_body(acc_ref, inv_ref, b_ref, o_ref):
    t = acc_ref[0] + acc_ref[1]
    o_ref[...] = inv_ref[0] * t + b_ref[...]


def _finalize(acc_pair, inv_col, bias):
    blk = 1264
    return pl.pallas_call(
        _fin_body,
        grid=(N_ACC // blk,),
        in_specs=[pl.BlockSpec((2, blk, D_FEAT), lambda i: (0, i, 0)),
                  pl.BlockSpec((1, blk, 1), lambda i: (0, i, 0)),
                  pl.BlockSpec((1, D_FEAT), lambda i: (0, 0))],
        out_specs=pl.BlockSpec((blk, D_FEAT), lambda i: (i, 0)),
        out_shape=jax.ShapeDtypeStruct((N_ACC, D_FEAT), jnp.float32),
    )(acc_pair, inv_col, bias.reshape(1, D_FEAT))


def _final_body(t_ref, dinv_ref, wmu_ref, bmu_ref, wlv_ref, blv_ref,
                mu_ref, lv_ref):
    t = t_ref[0] + t_ref[1]
    g = dinv_ref[...] * t
    s = jnp.concatenate(
        [jnp.sum(g[8 * k:8 * k + 8], axis=0, keepdims=True) for k in range(8)],
        axis=0)
    mu_ref[...] = jnp.dot(s, wmu_ref[...],
                          preferred_element_type=jnp.float32) + 8.0 * bmu_ref[...]
    lv_ref[...] = jnp.dot(s, wlv_ref[...],
                          preferred_element_type=jnp.float32) + 8.0 * blv_ref[...]


def _final(acc_t, dinv_t, Wmu, bmu, Wlv, blv):
    return pl.pallas_call(
        _final_body,
        out_shape=(jax.ShapeDtypeStruct((8, D_FEAT), jnp.float32),
                   jax.ShapeDtypeStruct((8, D_FEAT), jnp.float32)),
    )(acc_t, dinv_t, Wmu, bmu.reshape(1, D_FEAT), Wlv, blv.reshape(1, D_FEAT))


def kernel(x, edge_index, W0, b0, Wmu, bmu, Wlv, blv):
    node_idx = edge_index[0].astype(jnp.int32)
    he_idx = edge_index[1].astype(jnp.int32)
    pad = NW * HIST_PER_W - N_INC
    counts = [NC_CORE[w % 2] for w in range(NW)]
    starts = [0]
    for cw in counts:
        starts.append(starts[-1] + cw)

    def _blocks(idx, fill):
        flat = jnp.pad(idx, (0, TOTCH * CHUNK - N_INC), constant_values=fill)
        rows = []
        for w in range(NW):
            blk = flat[starts[w] * CHUNK:starts[w + 1] * CHUNK]
            blk = blk.reshape(counts[w], CHUNK)
            blk = jnp.pad(blk, ((0, MAXCH - counts[w]), (0, 0)),
                          constant_values=fill)
            rows.append(blk)
        return jnp.stack(rows)

    nig = _blocks(node_idx, 0)
    heg = _blocks(he_idx, 0)
    nis3 = _blocks(node_idx, N_NODES)
    hes3 = _blocks(he_idx, N_NODES)
    nis = jnp.pad(node_idx, (0, pad), constant_values=N_NODES)
    hes = jnp.pad(he_idx, (0, pad), constant_values=N_NODES)
    zeros = jnp.zeros((N_ACC, D_FEAT), jnp.float32)

    hist = _hist_pass(nis.reshape(NW, HIST_PER_W), hes.reshape(NW, HIST_PER_W))
    inv = _inverse_degrees(hist)
    inv3 = inv.reshape(2, N_ACC, 1)
    dinv_c = inv3[0:1]
    binv_c = inv3[1:2]

    xw = _matmul(x, W0)
    acc_e0 = _segsum_pass(xw, nig, hes3, zeros)
    out_e0 = _finalize(acc_e0, binv_c, jnp.zeros((D_FEAT,), jnp.float32))
    acc_h = _segsum_pass(out_e0[:N_NODES], heg, nis3, zeros)
    h = _finalize(acc_h, dinv_c, b0)
    acc_e1 = _segsum_pass(h[:N_NODES], nig, hes3, zeros)
    out_e1 = _finalize(acc_e1, binv_c, jnp.zeros((D_FEAT,), jnp.float32))

    cap = NW * TCAPCH * CHUNK
    mask = (node_idx < 8000) & ((node_idx % 1000) < 8)
    slots = (node_idx // 1000) * 8 + (node_idx % 1000)
    sel = jnp.nonzero(mask, size=cap, fill_value=N_INC)[0]
    gsel = jnp.append(he_idx, 0)[sel]
    ssel = jnp.append(slots, 64)[sel]
    gblk = gsel.reshape(TCAPCH, NW, CHUNK).transpose(1, 0, 2)
    sblk = ssel.reshape(TCAPCH, NW, CHUNK).transpose(1, 0, 2)
    nch_real = (jnp.sum(mask.astype(jnp.int32)) + CHUNK - 1) // CHUNK
    trips = jnp.clip((nch_real - jnp.arange(NW, dtype=jnp.int32) + NW - 1)
                     // NW, 0, TCAPCH).astype(jnp.int32)
    acc_g = _segsum_tgt_pass(out_e1[:N_NODES], gblk, sblk, trips, zeros)

    tgt = jnp.asarray(TARGETS, jnp.int32)
    acc_t = acc_g[:, :64, :]
    dinv_t = inv[0, tgt].reshape(64, 1)
    mu_p, lv_p = _final(acc_t, dinv_t, Wmu, bmu, Wlv, blv)
    return (mu_p, lv_p)


# speedup vs baseline: 4.2237x; 4.2237x over previous
import functools

import jax
import jax.numpy as jnp
from jax import lax
from jax.experimental import pallas as pl
from jax.experimental.pallas import tpu as pltpu
from jax.experimental.pallas import tpu_sc as plsc

N_NODES = 10000
N_INC = 320000
D_FEAT = 128
N_ACC = 10112
ROWS_PER_TILE = N_ACC // 16
NW = 32
CHUNK = 128
NC_CORE = (107, 50)
MAXCH = max(NC_CORE)
TOTCH = 16 * (NC_CORE[0] + NC_CORE[1])
HIST_PER_W = 10240
GROUPS = [[1, 2, 3, 4, 5, 6, 7, 8],
          [1001, 1002, 1003, 1004, 1005, 1006, 1007, 1008],
          [2001, 2002, 2003, 2004, 2005, 2006, 2007, 2008],
          [3001, 3002, 3003, 3004, 3005, 3006, 3007, 3008],
          [4001, 4002, 4003, 4004, 4005, 4006, 4007, 4008],
          [5001, 5002, 5003, 5004, 5005, 5006, 5007, 5008],
          [6001, 6002, 6003, 6004, 6005, 6006, 6007, 6008],
          [7001, 7002, 7003, 7004, 7005, 7006, 7007, 7008]]
TARGETS = [g - 1 for grp in GROUPS for g in grp]

_MESH = plsc.VectorSubcoreMesh(core_axis_name="c", subcore_axis_name="s")


TCAPCH = HIST_PER_W // 128


@functools.partial(
    pl.kernel,
    mesh=_MESH,
    out_type=(jax.ShapeDtypeStruct((2, 16, 2, N_ACC), jnp.float32),
              jax.ShapeDtypeStruct((2, 16, 2, HIST_PER_W), jnp.int32),
              jax.ShapeDtypeStruct((2, 16, 16), jnp.int32)),
    scratch_types=[
        pltpu.VMEM((HIST_PER_W,), jnp.int32),
        pltpu.VMEM((HIST_PER_W,), jnp.int32),
        pltpu.VMEM((N_ACC,), jnp.float32),
        pltpu.VMEM((N_ACC,), jnp.float32),
        pltpu.VMEM((HIST_PER_W,), jnp.int32),
        pltpu.VMEM((HIST_PER_W,), jnp.int32),
        pltpu.VMEM((16,), jnp.int32),
    ],
    compiler_params=pltpu.CompilerParams(needs_layout_passes=False),
)
def _hist_pass(nidx, hidx, out, lists, ocnt, nidx_v, hidx_v, hn, he,
               lg, ls, cbuf):
    c = lax.axis_index("c")
    s = lax.axis_index("s")
    w = s * 2 + c
    pltpu.sync_copy(nidx.at[w], nidx_v)
    pltpu.sync_copy(hidx.at[w], hidx_v)

    def zbody(i, carry):
        z = jnp.zeros((16,), jnp.float32)
        hn[pl.ds(16 * i, 16)] = z
        he[pl.ds(16 * i, 16)] = z
        return carry

    lax.fori_loop(0, N_ACC // 16, zbody, 0)

    def zbody2(i, carry):
        lg[pl.ds(16 * i, 16)] = jnp.zeros((16,), jnp.int32)
        ls[pl.ds(16 * i, 16)] = jnp.full((16,), 64, jnp.int32)
        return carry

    lax.fori_loop(0, HIST_PER_W // 16, zbody2, 0)

    def hbody(i, tcnt):
        v = nidx_v[pl.ds(16 * i, 16)]
        cnt, last = plsc.scan_count(v)
        plsc.addupdate_scatter(hn, [v], cnt.astype(jnp.float32), mask=last)
        u = hidx_v[pl.ds(16 * i, 16)]
        cnt2, last2 = plsc.scan_count(u)
        plsc.addupdate_scatter(he, [u], cnt2.astype(jnp.float32), mask=last2)
        q = v // 1000
        r = v - q * 1000
        m = (v < 8000) & (r < 8)
        slot = q * 8 + r
        plsc.store_compressed(lg.at[pl.ds(tcnt, 16)], u, mask=m)
        plsc.store_compressed(ls.at[pl.ds(tcnt, 16)], slot, mask=m)
        return tcnt + jnp.max(plsc.all_reduce_population_count(m))

    tcnt = lax.fori_loop(0, HIST_PER_W // 16, hbody, jnp.int32(0))
    cbuf[...] = jnp.broadcast_to(tcnt, (16,))
    pltpu.sync_copy(hn, out.at[c, s, 0])
    pltpu.sync_copy(he, out.at[c, s, 1])
    pltpu.sync_copy(lg, lists.at[c, s, 0])
    pltpu.sync_copy(ls, lists.at[c, s, 1])
    pltpu.sync_copy(cbuf, ocnt.at[c, s])


def _inv_body(h_ref, o_ref):
    t = jnp.sum(h_ref[...], axis=(0, 1))
    o_ref[...] = jnp.where(t > 0, 1.0 / jnp.where(t > 0, t, 1.0), 0.0)


def _inverse_degrees(hist):
    return pl.pallas_call(
        _inv_body,
        out_shape=jax.ShapeDtypeStruct((2, N_ACC), jnp.float32),
    )(hist)


def _mm_body(x_ref, w_ref, o_ref):
    o_ref[...] = jnp.dot(x_ref[...], w_ref[...],
                         preferred_element_type=jnp.float32)


def _matmul(x, W):
    blk = 1000
    return pl.pallas_call(
        _mm_body,
        grid=(N_NODES // blk,),
        in_specs=[pl.BlockSpec((blk, D_FEAT), lambda i: (i, 0)),
                  pl.BlockSpec((D_FEAT, D_FEAT), lambda i: (0, 0))],
        out_specs=pl.BlockSpec((blk, D_FEAT), lambda i: (i, 0)),
        out_shape=jax.ShapeDtypeStruct((N_NODES, D_FEAT), jnp.float32),
    )(x, W)


@functools.partial(
    pl.kernel,
    mesh=_MESH,
    out_type=jax.ShapeDtypeStruct((2, N_ACC, D_FEAT), jnp.float32),
    scratch_types=[
        pltpu.VMEM((MAXCH, CHUNK), jnp.int32),
        pltpu.VMEM((MAXCH, CHUNK), jnp.int32),
        pltpu.VMEM((CHUNK, D_FEAT), jnp.float32),
        pltpu.VMEM_SHARED((N_ACC, D_FEAT), jnp.float32),
        pltpu.SemaphoreType.DMA,
    ],
)
def _segsum_pass(src, idxg, idxs, zeros, out, idxg_v, idxs_v, rows, acc,
                 gsem):
    c = lax.axis_index("c")
    s = lax.axis_index("s")
    w = s * 2 + c
    pltpu.sync_copy(zeros.at[pl.ds(s * ROWS_PER_TILE, ROWS_PER_TILE)],
                    acc.at[pl.ds(s * ROWS_PER_TILE, ROWS_PER_TILE)])
    pltpu.sync_copy(idxg.at[w], idxg_v)
    pltpu.sync_copy(idxs.at[w], idxs_v)
    plsc.subcore_barrier()

    def body(j, carry):
        pltpu.async_copy(src.at[idxg_v.at[j]], rows, gsem).wait()
        pltpu.sync_copy(rows, acc.at[idxs_v.at[j]], add=True)
        return carry

    nmine = jnp.where(c == 0, NC_CORE[0], NC_CORE[1])
    lax.fori_loop(0, nmine, body, 0)
    plsc.subcore_barrier()
    pltpu.sync_copy(acc.at[pl.ds(s * ROWS_PER_TILE, ROWS_PER_TILE)],
                    out.at[c, pl.ds(s * ROWS_PER_TILE, ROWS_PER_TILE)])


@functools.partial(
    pl.kernel,
    mesh=_MESH,
    out_type=jax.ShapeDtypeStruct((2, 128, D_FEAT), jnp.float32),
    scratch_types=[
        pltpu.VMEM((HIST_PER_W,), jnp.int32),
        pltpu.VMEM((HIST_PER_W,), jnp.int32),
        pltpu.VMEM((16,), jnp.int32),
        pltpu.VMEM((CHUNK,), jnp.int32),
        pltpu.VMEM((CHUNK,), jnp.int32),
        pltpu.VMEM((CHUNK, D_FEAT), jnp.float32),
        pltpu.VMEM_SHARED((128, D_FEAT), jnp.float32),
        pltpu.SemaphoreType.DMA,
    ],
    compiler_params=pltpu.CompilerParams(needs_layout_passes=False),
)
def _segsum_tgt_pass(src, lists, cnts, zeros, out, lgv, lsv, cnt_v,
                     gbuf, sbuf, rows, acc, gsem):
    c = lax.axis_index("c")
    s = lax.axis_index("s")
    pltpu.sync_copy(zeros.at[pl.ds(s * 8, 8)], acc.at[pl.ds(s * 8, 8)])
    pltpu.sync_copy(lists.at[c, s, 0], lgv)
    pltpu.sync_copy(lists.at[c, s, 1], lsv)
    pltpu.sync_copy(cnts.at[c, s], cnt_v)
    plsc.subcore_barrier()

    nmine = (jnp.max(cnt_v[...]) + CHUNK - 1) // CHUNK

    def body(j, carry):
        for i in range(CHUNK // 16):
            gbuf[pl.ds(16 * i, 16)] = lgv[pl.ds(j * CHUNK + 16 * i, 16)]
            sbuf[pl.ds(16 * i, 16)] = lsv[pl.ds(j * CHUNK + 16 * i, 16)]
        pltpu.async_copy(src.at[gbuf], rows, gsem).wait()
        pltpu.sync_copy(rows, acc.at[sbuf], add=True)
        return carry

    lax.fori_loop(0, nmine, body, 0)
    plsc.subcore_barrier()
    pltpu.sync_copy(acc.at[pl.ds(s * 8, 8)], out.at[c, pl.ds(s * 8, 8)])


def _fin_body(acc_ref, inv_ref, b_ref, o_ref):
    t = acc_ref[0] + acc_ref[1]
    o_ref[...] = inv_ref[0] * t + b_ref[...]


def _finalize(acc_pair, inv_col, bias):
    blk = 1264
    return pl.pallas_call(
        _fin_body,
        grid=(N_ACC // blk,),
        in_specs=[pl.BlockSpec((2, blk, D_FEAT), lambda i: (0, i, 0)),
                  pl.BlockSpec((1, blk, 1), lambda i: (0, i, 0)),
                  pl.BlockSpec((1, D_FEAT), lambda i: (0, 0))],
        out_specs=pl.BlockSpec((blk, D_FEAT), lambda i: (i, 0)),
        out_shape=jax.ShapeDtypeStruct((N_ACC, D_FEAT), jnp.float32),
    )(acc_pair, inv_col, bias.reshape(1, D_FEAT))


def _final_body(t_ref, dinv_ref, wmu_ref, bmu_ref, wlv_ref, blv_ref,
                mu_ref, lv_ref):
    t = t_ref[0] + t_ref[1]
    g = dinv_ref[...] * t
    s = jnp.concatenate(
        [jnp.sum(g[8 * k:8 * k + 8], axis=0, keepdims=True) for k in range(8)],
        axis=0)
    mu_ref[...] = jnp.dot(s, wmu_ref[...],
                          preferred_element_type=jnp.float32) + 8.0 * bmu_ref[...]
    lv_ref[...] = jnp.dot(s, wlv_ref[...],
                          preferred_element_type=jnp.float32) + 8.0 * blv_ref[...]


def _final(acc_t, dinv_t, Wmu, bmu, Wlv, blv):
    return pl.pallas_call(
        _final_body,
        out_shape=(jax.ShapeDtypeStruct((8, D_FEAT), jnp.float32),
                   jax.ShapeDtypeStruct((8, D_FEAT), jnp.float32)),
    )(acc_t, dinv_t, Wmu, bmu.reshape(1, D_FEAT), Wlv, blv.reshape(1, D_FEAT))


def kernel(x, edge_index, W0, b0, Wmu, bmu, Wlv, blv):
    node_idx = edge_index[0].astype(jnp.int32)
    he_idx = edge_index[1].astype(jnp.int32)
    pad = NW * HIST_PER_W - N_INC
    counts = [NC_CORE[w % 2] for w in range(NW)]
    starts = [0]
    for cw in counts:
        starts.append(starts[-1] + cw)

    def _blocks(idx, fill):
        flat = jnp.pad(idx, (0, TOTCH * CHUNK - N_INC), constant_values=fill)
        rows = []
        for w in range(NW):
            blk = flat[starts[w] * CHUNK:starts[w + 1] * CHUNK]
            blk = blk.reshape(counts[w], CHUNK)
            blk = jnp.pad(blk, ((0, MAXCH - counts[w]), (0, 0)),
                          constant_values=fill)
            rows.append(blk)
        return jnp.stack(rows)

    nig = _blocks(node_idx, 0)
    heg = _blocks(he_idx, 0)
    nis3 = _blocks(node_idx, N_NODES)
    hes3 = _blocks(he_idx, N_NODES)
    nis = jnp.pad(node_idx, (0, pad), constant_values=N_NODES)
    hes = jnp.pad(he_idx, (0, pad), constant_values=N_NODES)
    zeros = jnp.zeros((N_ACC, D_FEAT), jnp.float32)

    hist, tlists, tcnts = _hist_pass(nis.reshape(NW, HIST_PER_W),
                                     hes.reshape(NW, HIST_PER_W))
    inv = _inverse_degrees(hist)
    inv3 = inv.reshape(2, N_ACC, 1)
    dinv_c = inv3[0:1]
    binv_c = inv3[1:2]

    xw = _matmul(x, W0)
    acc_e0 = _segsum_pass(xw, nig, hes3, zeros)
    out_e0 = _finalize(acc_e0, binv_c, jnp.zeros((D_FEAT,), jnp.float32))
    acc_h = _segsum_pass(out_e0[:N_NODES], heg, nis3, zeros)
    h = _finalize(acc_h, dinv_c, b0)
    acc_e1 = _segsum_pass(h[:N_NODES], nig, hes3, zeros)
    out_e1 = _finalize(acc_e1, binv_c, jnp.zeros((D_FEAT,), jnp.float32))

    acc_g = _segsum_tgt_pass(out_e1[:N_NODES], tlists, tcnts, zeros)

    tgt = jnp.asarray(TARGETS, jnp.int32)
    acc_t = acc_g[:, :64, :]
    dinv_t = inv[0, tgt].reshape(64, 1)
    mu_p, lv_p = _final(acc_t, dinv_t, Wmu, bmu, Wlv, blv)
    return (mu_p, lv_p)


# speedup vs baseline: 4.5087x; 1.0675x over previous
import functools

import jax
import jax.numpy as jnp
from jax import lax
from jax.experimental import pallas as pl
from jax.experimental.pallas import tpu as pltpu
from jax.experimental.pallas import tpu_sc as plsc

N_NODES = 10000
N_INC = 320000
D_FEAT = 128
N_ACC = 10112
ROWS_PER_TILE = N_ACC // 16
NW = 32
CHUNK = 128
NC_CORE = (97, 60)
MAXCH = max(NC_CORE)
TOTCH = 16 * (NC_CORE[0] + NC_CORE[1])
HIST_PER_W = 10240
GROUPS = [[1, 2, 3, 4, 5, 6, 7, 8],
          [1001, 1002, 1003, 1004, 1005, 1006, 1007, 1008],
          [2001, 2002, 2003, 2004, 2005, 2006, 2007, 2008],
          [3001, 3002, 3003, 3004, 3005, 3006, 3007, 3008],
          [4001, 4002, 4003, 4004, 4005, 4006, 4007, 4008],
          [5001, 5002, 5003, 5004, 5005, 5006, 5007, 5008],
          [6001, 6002, 6003, 6004, 6005, 6006, 6007, 6008],
          [7001, 7002, 7003, 7004, 7005, 7006, 7007, 7008]]
TARGETS = [g - 1 for grp in GROUPS for g in grp]

_MESH = plsc.VectorSubcoreMesh(core_axis_name="c", subcore_axis_name="s")


TCAPCH = HIST_PER_W // 128


@functools.partial(
    pl.kernel,
    mesh=_MESH,
    out_type=(jax.ShapeDtypeStruct((2, 16, 2, N_ACC), jnp.float32),
              jax.ShapeDtypeStruct((2, 16, 2, HIST_PER_W), jnp.int32),
              jax.ShapeDtypeStruct((2, 16, 16), jnp.int32)),
    scratch_types=[
        pltpu.VMEM((HIST_PER_W,), jnp.int32),
        pltpu.VMEM((HIST_PER_W,), jnp.int32),
        pltpu.VMEM((N_ACC,), jnp.float32),
        pltpu.VMEM((N_ACC,), jnp.float32),
        pltpu.VMEM((HIST_PER_W,), jnp.int32),
        pltpu.VMEM((HIST_PER_W,), jnp.int32),
        pltpu.VMEM((16,), jnp.int32),
    ],
    compiler_params=pltpu.CompilerParams(needs_layout_passes=False),
)
def _hist_pass(nidx, hidx, out, lists, ocnt, nidx_v, hidx_v, hn, he,
               lg, ls, cbuf):
    c = lax.axis_index("c")
    s = lax.axis_index("s")
    w = s * 2 + c
    pltpu.sync_copy(nidx.at[w], nidx_v)
    pltpu.sync_copy(hidx.at[w], hidx_v)

    def zbody(i, carry):
        z = jnp.zeros((16,), jnp.float32)
        hn[pl.ds(16 * i, 16)] = z
        he[pl.ds(16 * i, 16)] = z
        return carry

    lax.fori_loop(0, N_ACC // 16, zbody, 0)

    def zbody2(i, carry):
        lg[pl.ds(16 * i, 16)] = jnp.zeros((16,), jnp.int32)
        ls[pl.ds(16 * i, 16)] = jnp.full((16,), 64, jnp.int32)
        return carry

    lax.fori_loop(0, HIST_PER_W // 16, zbody2, 0)

    def hbody(i, tcnt):
        v = nidx_v[pl.ds(16 * i, 16)]
        cnt, last = plsc.scan_count(v)
        plsc.addupdate_scatter(hn, [v], cnt.astype(jnp.float32), mask=last)
        u = hidx_v[pl.ds(16 * i, 16)]
        cnt2, last2 = plsc.scan_count(u)
        plsc.addupdate_scatter(he, [u], cnt2.astype(jnp.float32), mask=last2)
        q = v // 1000
        r = v - q * 1000
        m = (v < 8000) & (r < 8)
        slot = q * 8 + r
        plsc.store_compressed(lg.at[pl.ds(tcnt, 16)], u, mask=m)
        plsc.store_compressed(ls.at[pl.ds(tcnt, 16)], slot, mask=m)
        return tcnt + jnp.max(plsc.all_reduce_population_count(m))

    tcnt = lax.fori_loop(0, HIST_PER_W // 16, hbody, jnp.int32(0))
    cbuf[...] = jnp.broadcast_to(tcnt, (16,))
    pltpu.sync_copy(hn, out.at[c, s, 0])
    pltpu.sync_copy(he, out.at[c, s, 1])
    pltpu.sync_copy(lg, lists.at[c, s, 0])
    pltpu.sync_copy(ls, lists.at[c, s, 1])
    pltpu.sync_copy(cbuf, ocnt.at[c, s])


def _inv_body(h_ref, o_ref):
    t = jnp.sum(h_ref[...], axis=(0, 1))
    o_ref[...] = jnp.where(t > 0, 1.0 / jnp.where(t > 0, t, 1.0), 0.0)


def _inverse_degrees(hist):
    return pl.pallas_call(
        _inv_body,
        out_shape=jax.ShapeDtypeStruct((2, N_ACC), jnp.float32),
    )(hist)


def _mm_body(x_ref, w_ref, o_ref):
    o_ref[...] = jnp.dot(x_ref[...], w_ref[...],
                         preferred_element_type=jnp.float32)


def _matmul(x, W):
    blk = 1000
    return pl.pallas_call(
        _mm_body,
        grid=(N_NODES // blk,),
        in_specs=[pl.BlockSpec((blk, D_FEAT), lambda i: (i, 0)),
                  pl.BlockSpec((D_FEAT, D_FEAT), lambda i: (0, 0))],
        out_specs=pl.BlockSpec((blk, D_FEAT), lambda i: (i, 0)),
        out_shape=jax.ShapeDtypeStruct((N_NODES, D_FEAT), jnp.float32),
    )(x, W)


@functools.partial(
    pl.kernel,
    mesh=_MESH,
    out_type=jax.ShapeDtypeStruct((2, N_ACC, D_FEAT), jnp.float32),
    scratch_types=[
        pltpu.VMEM((MAXCH, CHUNK), jnp.int32),
        pltpu.VMEM((MAXCH, CHUNK), jnp.int32),
        pltpu.VMEM((CHUNK, D_FEAT), jnp.float32),
        pltpu.VMEM_SHARED((N_ACC, D_FEAT), jnp.float32),
        pltpu.SemaphoreType.DMA,
    ],
)
def _segsum_pass(src, idxg, idxs, zeros, out, idxg_v, idxs_v, rows, acc,
                 gsem):
    c = lax.axis_index("c")
    s = lax.axis_index("s")
    w = s * 2 + c
    pltpu.sync_copy(zeros.at[pl.ds(s * ROWS_PER_TILE, ROWS_PER_TILE)],
                    acc.at[pl.ds(s * ROWS_PER_TILE, ROWS_PER_TILE)])
    pltpu.sync_copy(idxg.at[w], idxg_v)
    pltpu.sync_copy(idxs.at[w], idxs_v)
    plsc.subcore_barrier()

    def body(j, carry):
        pltpu.async_copy(src.at[idxg_v.at[j]], rows, gsem).wait()
        pltpu.sync_copy(rows, acc.at[idxs_v.at[j]], add=True)
        return carry

    nmine = jnp.where(c == 0, NC_CORE[0], NC_CORE[1])
    lax.fori_loop(0, nmine, body, 0)
    plsc.subcore_barrier()
    pltpu.sync_copy(acc.at[pl.ds(s * ROWS_PER_TILE, ROWS_PER_TILE)],
                    out.at[c, pl.ds(s * ROWS_PER_TILE, ROWS_PER_TILE)])


@functools.partial(
    pl.kernel,
    mesh=_MESH,
    out_type=jax.ShapeDtypeStruct((2, 128, D_FEAT), jnp.float32),
    scratch_types=[
        pltpu.VMEM((16,), jnp.int32),
        pltpu.VMEM((CHUNK,), jnp.int32),
        pltpu.VMEM((CHUNK,), jnp.int32),
        pltpu.VMEM((CHUNK, D_FEAT), jnp.float32),
        pltpu.VMEM_SHARED((128, D_FEAT), jnp.float32),
        pltpu.SemaphoreType.DMA,
    ],
    compiler_params=pltpu.CompilerParams(needs_layout_passes=False),
)
def _segsum_tgt_pass(src, lists, cnts, zeros, out, cnt_v,
                     gbuf, sbuf, rows, acc, gsem):
    c = lax.axis_index("c")
    s = lax.axis_index("s")
    pltpu.sync_copy(zeros.at[pl.ds(s * 8, 8)], acc.at[pl.ds(s * 8, 8)])
    pltpu.sync_copy(cnts.at[c, s], cnt_v)
    plsc.subcore_barrier()

    nmine = (jnp.max(cnt_v[...]) + CHUNK - 1) // CHUNK

    def body(j, carry):
        pltpu.sync_copy(lists.at[c, s, 0, pl.ds(j * CHUNK, CHUNK)], gbuf)
        pltpu.sync_copy(lists.at[c, s, 1, pl.ds(j * CHUNK, CHUNK)], sbuf)
        pltpu.async_copy(src.at[gbuf], rows, gsem).wait()
        pltpu.sync_copy(rows, acc.at[sbuf], add=True)
        return carry

    lax.fori_loop(0, nmine, body, 0)
    plsc.subcore_barrier()
    pltpu.sync_copy(acc.at[pl.ds(s * 8, 8)], out.at[c, pl.ds(s * 8, 8)])


def _fin_body(acc_ref, inv_ref, b_ref, o_ref):
    t = acc_ref[0] + acc_ref[1]
    o_ref[...] = inv_ref[0] * t + b_ref[...]


def _finalize(acc_pair, inv_col, bias):
    blk = 1264
    return pl.pallas_call(
        _fin_body,
        grid=(N_ACC // blk,),
        in_specs=[pl.BlockSpec((2, blk, D_FEAT), lambda i: (0, i, 0)),
                  pl.BlockSpec((1, blk, 1), lambda i: (0, i, 0)),
                  pl.BlockSpec((1, D_FEAT), lambda i: (0, 0))],
        out_specs=pl.BlockSpec((blk, D_FEAT), lambda i: (i, 0)),
        out_shape=jax.ShapeDtypeStruct((N_ACC, D_FEAT), jnp.float32),
    )(acc_pair, inv_col, bias.reshape(1, D_FEAT))


def _final_body(t_ref, dinv_ref, wmu_ref, bmu_ref, wlv_ref, blv_ref,
                mu_ref, lv_ref):
    t = t_ref[0] + t_ref[1]
    g = dinv_ref[...] * t
    s = jnp.concatenate(
        [jnp.sum(g[8 * k:8 * k + 8], axis=0, keepdims=True) for k in range(8)],
        axis=0)
    mu_ref[...] = jnp.dot(s, wmu_ref[...],
                          preferred_element_type=jnp.float32) + 8.0 * bmu_ref[...]
    lv_ref[...] = jnp.dot(s, wlv_ref[...],
                          preferred_element_type=jnp.float32) + 8.0 * blv_ref[...]


def _final(acc_t, dinv_t, Wmu, bmu, Wlv, blv):
    return pl.pallas_call(
        _final_body,
        out_shape=(jax.ShapeDtypeStruct((8, D_FEAT), jnp.float32),
                   jax.ShapeDtypeStruct((8, D_FEAT), jnp.float32)),
    )(acc_t, dinv_t, Wmu, bmu.reshape(1, D_FEAT), Wlv, blv.reshape(1, D_FEAT))


def kernel(x, edge_index, W0, b0, Wmu, bmu, Wlv, blv):
    node_idx = edge_index[0].astype(jnp.int32)
    he_idx = edge_index[1].astype(jnp.int32)
    pad = NW * HIST_PER_W - N_INC
    counts = [NC_CORE[w % 2] for w in range(NW)]
    starts = [0]
    for cw in counts:
        starts.append(starts[-1] + cw)

    def _blocks(idx, fill):
        flat = jnp.pad(idx, (0, TOTCH * CHUNK - N_INC), constant_values=fill)
        rows = []
        for w in range(NW):
            blk = flat[starts[w] * CHUNK:starts[w + 1] * CHUNK]
            blk = blk.reshape(counts[w], CHUNK)
            blk = jnp.pad(blk, ((0, MAXCH - counts[w]), (0, 0)),
                          constant_values=fill)
            rows.append(blk)
        return jnp.stack(rows)

    nig = _blocks(node_idx, 0)
    heg = _blocks(he_idx, 0)
    nis3 = _blocks(node_idx, N_NODES)
    hes3 = _blocks(he_idx, N_NODES)
    nis = jnp.pad(node_idx, (0, pad), constant_values=N_NODES)
    hes = jnp.pad(he_idx, (0, pad), constant_values=N_NODES)
    zeros = jnp.zeros((N_ACC, D_FEAT), jnp.float32)

    hist, tlists, tcnts = _hist_pass(nis.reshape(NW, HIST_PER_W),
                                     hes.reshape(NW, HIST_PER_W))
    inv = _inverse_degrees(hist)
    inv3 = inv.reshape(2, N_ACC, 1)
    dinv_c = inv3[0:1]
    binv_c = inv3[1:2]

    xw = _matmul(x, W0)
    acc_e0 = _segsum_pass(xw, nig, hes3, zeros)
    out_e0 = _finalize(acc_e0, binv_c, jnp.zeros((D_FEAT,), jnp.float32))
    acc_h = _segsum_pass(out_e0[:N_NODES], heg, nis3, zeros)
    h = _finalize(acc_h, dinv_c, b0)
    acc_e1 = _segsum_pass(h[:N_NODES], nig, hes3, zeros)
    out_e1 = _finalize(acc_e1, binv_c, jnp.zeros((D_FEAT,), jnp.float32))

    acc_g = _segsum_tgt_pass(out_e1[:N_NODES], tlists, tcnts, zeros)

    tgt = jnp.asarray(TARGETS, jnp.int32)
    acc_t = acc_g[:, :64, :]
    dinv_t = inv[0, tgt].reshape(64, 1)
    mu_p, lv_p = _final(acc_t, dinv_t, Wmu, bmu, Wlv, blv)
    return (mu_p, lv_p)
